# Initial kernel scaffold; baseline (speedup 1.0000x reference)
import jax
import jax.numpy as jnp
from jax.experimental import pallas as pl


def kernel(x, W, b, gamma, beta):
    raise NotImplementedError("write your pallas kernel here")


# speedup vs baseline: 2.9697x; 2.9697x over previous
import jax
import jax.numpy as jnp
from jax.experimental import pallas as pl

B, C, N, OC, K = 4, 64, 4096, 128, 16
R = 128

_HIGH = jax.lax.Precision.HIGHEST


def _knn_conv_kernel(qt_ref, xk_ref, xt_ref, w_ref, b_ref,
                     h_ref, sum_ref, sumsq_ref):
    qt = qt_ref[0]
    xk = xk_ref[0]
    xt = xt_ref[0]

    dot = jax.lax.dot_general(qt, xk, (((1,), (0,)), ((), ())),
                              preferred_element_type=jnp.float32)
    xx = jnp.sum(xk * xk, axis=0, keepdims=True)
    s = 2.0 * dot - xx

    col = jax.lax.broadcasted_iota(jnp.int32, (R, N), 1)
    xm = jnp.full((R, C), -jnp.inf, dtype=jnp.float32)
    for _ in range(K):
        m = jnp.max(s, axis=1, keepdims=True)
        idx = jnp.min(jnp.where(s == m, col, N), axis=1,
                      keepdims=True)
        onehot = (col == idx)
        feat = jax.lax.dot_general(onehot.astype(jnp.float32), xt,
                                   (((1,), (0,)), ((), ())),
                                   preferred_element_type=jnp.float32,
                                   precision=_HIGH)
        xm = jnp.maximum(xm, feat)
        s = jnp.where(onehot, -jnp.inf, s)

    h = jax.lax.dot_general(xm, w_ref[...], (((1,), (1,)), ((), ())),
                            preferred_element_type=jnp.float32)
    h = h + b_ref[...]
    h_ref[0] = h

    first = jnp.logical_and(pl.program_id(0) == 0, pl.program_id(1) == 0)

    @pl.when(first)
    def _init():
        sum_ref[...] = jnp.zeros((1, OC), jnp.float32)
        sumsq_ref[...] = jnp.zeros((1, OC), jnp.float32)

    sum_ref[...] += jnp.sum(h, axis=0, keepdims=True)
    sumsq_ref[...] += jnp.sum(h * h, axis=0, keepdims=True)


def _bn_relu_kernel(h_ref, sum_ref, sumsq_ref, gamma_ref, beta_ref, out_ref):
    h = h_ref[0]
    cnt = jnp.float32(B * N)
    mean = sum_ref[...] / cnt
    var = sumsq_ref[...] / cnt - mean * mean
    inv = 1.0 / jnp.sqrt(var + 1e-5)
    y = (h - mean) * inv * gamma_ref[...] + beta_ref[...]
    y = jnp.maximum(y, 0.0)
    out_ref[0] = jnp.transpose(y)


def kernel(x, W, b, gamma, beta):
    xt = jnp.transpose(x, (0, 2, 1))
    b2 = b.reshape(1, OC)
    gamma2 = gamma.reshape(1, OC)
    beta2 = beta.reshape(1, OC)

    grid = (B, N // R)
    h, s1, s2 = pl.pallas_call(
        _knn_conv_kernel,
        grid=grid,
        in_specs=[
            pl.BlockSpec((1, R, C), lambda bb, rr: (bb, rr, 0)),
            pl.BlockSpec((1, C, N), lambda bb, rr: (bb, 0, 0)),
            pl.BlockSpec((1, N, C), lambda bb, rr: (bb, 0, 0)),
            pl.BlockSpec((OC, C), lambda bb, rr: (0, 0)),
            pl.BlockSpec((1, OC), lambda bb, rr: (0, 0)),
        ],
        out_specs=[
            pl.BlockSpec((1, R, OC), lambda bb, rr: (bb, rr, 0)),
            pl.BlockSpec((1, OC), lambda bb, rr: (0, 0)),
            pl.BlockSpec((1, OC), lambda bb, rr: (0, 0)),
        ],
        out_shape=[
            jax.ShapeDtypeStruct((B, N, OC), jnp.float32),
            jax.ShapeDtypeStruct((1, OC), jnp.float32),
            jax.ShapeDtypeStruct((1, OC), jnp.float32),
        ],
    )(xt, x, xt, W, b2)

    out = pl.pallas_call(
        _bn_relu_kernel,
        grid=grid,
        in_specs=[
            pl.BlockSpec((1, R, OC), lambda bb, rr: (bb, rr, 0)),
            pl.BlockSpec((1, OC), lambda bb, rr: (0, 0)),
            pl.BlockSpec((1, OC), lambda bb, rr: (0, 0)),
            pl.BlockSpec((1, OC), lambda bb, rr: (0, 0)),
            pl.BlockSpec((1, OC), lambda bb, rr: (0, 0)),
        ],
        out_specs=pl.BlockSpec((1, OC, R), lambda bb, rr: (bb, 0, rr)),
        out_shape=jax.ShapeDtypeStruct((B, OC, N), jnp.float32),
    )(h, s1, s2, gamma2, beta2)
    return out


# speedup vs baseline: 12.2842x; 4.1365x over previous
import functools

import jax
import jax.numpy as jnp
from jax.experimental import pallas as pl
from jax.experimental.pallas import tpu as pltpu
from jax.experimental.pallas import tpu_sc as plsc

B, C, N, OC, K = 4, 64, 4096, 128, 16
R = 128
RC = 512

_HIGH = jax.lax.Precision.HIGHEST

NC, NS = 2, 16
NW = NC * NS
PPW = (B * N) // NW
GP = 8
GIDX = GP * K
GPW = PPW // GP


def _knn_topk_kernel(qt_ref, xk_ref, idx_ref):
    qt = qt_ref[0]
    xk = xk_ref[0]

    dot = jax.lax.dot_general(qt, xk, (((1,), (0,)), ((), ())),
                              preferred_element_type=jnp.float32)
    xx = jnp.sum(xk * xk, axis=0, keepdims=True)
    s = 2.0 * dot - xx

    col = jax.lax.broadcasted_iota(jnp.int32, (R, N), 1)
    picks = []
    for _ in range(K):
        m = jnp.max(s, axis=1, keepdims=True)
        idx = jnp.min(jnp.where(s == m, col, N), axis=1,
                      keepdims=True)
        picks.append(idx)
        s = jnp.where(col == idx, -jnp.inf, s)

    idxs = jnp.concatenate(picks, axis=1)
    idx_ref[0] = idxs + pl.program_id(0) * N


def _gather_max_sc(table_hbm, idx_hbm, out_hbm, idx_v, rows_v, out_v,
                   sem0, sem1):
    wid = jax.lax.axis_index("s") * NC + jax.lax.axis_index("c")
    ibase = wid * (PPW * K)
    obase = wid * (PPW // 2)
    sems = (sem0, sem1)

    def issue(c, slot):
        pltpu.sync_copy(idx_hbm.at[pl.ds(ibase + c * GIDX, GIDX)],
                        idx_v.at[slot])
        pltpu.make_async_copy(table_hbm.at[idx_v.at[slot]],
                              rows_v.at[slot], sems[slot]).start()

    def consume(slot):
        pltpu.make_async_copy(table_hbm.at[idx_v.at[slot]],
                              rows_v.at[slot], sems[slot]).wait()
        for p in range(GP):
            for g in range(4):
                r0 = p * K
                acc = rows_v[slot, r0, pl.ds(g * 16, 16)]
                for j in range(1, K):
                    acc = jnp.maximum(
                        acc, rows_v[slot, r0 + j, pl.ds(g * 16, 16)])
                out_v[slot * (GP // 2) + p // 2,
                      pl.ds((p % 2) * C + g * 16, 16)] = acc

    issue(0, 0)

    def body(i, carry):
        c0 = 2 * i
        issue(c0 + 1, 1)
        consume(0)

        @pl.when(i < GPW // 2 - 1)
        def _():
            issue(c0 + 2, 0)

        consume(1)
        pltpu.sync_copy(out_v,
                        out_hbm.at[pl.ds(obase + i * GP, GP)])
        return carry

    jax.lax.fori_loop(0, GPW // 2, body, 0)


def _conv_kernel(xm_ref, w_ref, b_ref, h_ref, sum_ref, sumsq_ref):
    xm = xm_ref[0]
    h = jax.lax.dot_general(xm, w_ref[...], (((1,), (1,)), ((), ())),
                            preferred_element_type=jnp.float32)
    h = h + b_ref[...]
    h_ref[0] = h

    first = jnp.logical_and(pl.program_id(0) == 0, pl.program_id(1) == 0)

    @pl.when(first)
    def _init():
        sum_ref[...] = jnp.zeros((1, OC), jnp.float32)
        sumsq_ref[...] = jnp.zeros((1, OC), jnp.float32)

    sum_ref[...] += jnp.sum(h, axis=0, keepdims=True)
    sumsq_ref[...] += jnp.sum(h * h, axis=0, keepdims=True)


def _bn_relu_kernel(h_ref, sum_ref, sumsq_ref, gamma_ref, beta_ref, out_ref):
    h = h_ref[0]
    cnt = jnp.float32(B * N)
    mean = sum_ref[...] / cnt
    var = sumsq_ref[...] / cnt - mean * mean
    inv = 1.0 / jnp.sqrt(var + 1e-5)
    y = (h - mean) * inv * gamma_ref[...] + beta_ref[...]
    y = jnp.maximum(y, 0.0)
    out_ref[0] = jnp.transpose(y)


def kernel(x, W, b, gamma, beta):
    xt = jnp.transpose(x, (0, 2, 1))
    b2 = b.reshape(1, OC)
    gamma2 = gamma.reshape(1, OC)
    beta2 = beta.reshape(1, OC)

    idx = pl.pallas_call(
        _knn_topk_kernel,
        grid=(B, N // R),
        in_specs=[
            pl.BlockSpec((1, R, C), lambda bb, rr: (bb, rr, 0)),
            pl.BlockSpec((1, C, N), lambda bb, rr: (bb, 0, 0)),
        ],
        out_specs=pl.BlockSpec((1, R, K), lambda bb, rr: (bb, rr, 0)),
        out_shape=jax.ShapeDtypeStruct((B, N, K), jnp.int32),
    )(xt, x)

    table = jnp.concatenate(
        [xt.reshape(B * N, C), jnp.zeros((B * N, C), jnp.float32)], axis=1)
    idx_flat = idx.reshape(B * N * K)

    gather_max = functools.partial(
        pl.kernel,
        mesh=plsc.VectorSubcoreMesh(core_axis_name="c", subcore_axis_name="s"),
        out_type=jax.ShapeDtypeStruct((B * N // 2, 2 * C), jnp.float32),
        scratch_types=[
            pltpu.VMEM((2, GIDX), jnp.int32),
            pltpu.VMEM((2, GIDX, 2 * C), jnp.float32),
            pltpu.VMEM((GP, 2 * C), jnp.float32),
            pltpu.SemaphoreType.DMA,
            pltpu.SemaphoreType.DMA,
        ],
    )(_gather_max_sc)
    xm = gather_max(table, idx_flat).reshape(B, N, C)

    grid = (B, N // RC)
    h, s1, s2 = pl.pallas_call(
        _conv_kernel,
        grid=grid,
        in_specs=[
            pl.BlockSpec((1, RC, C), lambda bb, rr: (bb, rr, 0)),
            pl.BlockSpec((OC, C), lambda bb, rr: (0, 0)),
            pl.BlockSpec((1, OC), lambda bb, rr: (0, 0)),
        ],
        out_specs=[
            pl.BlockSpec((1, RC, OC), lambda bb, rr: (bb, rr, 0)),
            pl.BlockSpec((1, OC), lambda bb, rr: (0, 0)),
            pl.BlockSpec((1, OC), lambda bb, rr: (0, 0)),
        ],
        out_shape=[
            jax.ShapeDtypeStruct((B, N, OC), jnp.float32),
            jax.ShapeDtypeStruct((1, OC), jnp.float32),
            jax.ShapeDtypeStruct((1, OC), jnp.float32),
        ],
    )(xm, W, b2)

    out = pl.pallas_call(
        _bn_relu_kernel,
        grid=grid,
        in_specs=[
            pl.BlockSpec((1, RC, OC), lambda bb, rr: (bb, rr, 0)),
            pl.BlockSpec((1, OC), lambda bb, rr: (0, 0)),
            pl.BlockSpec((1, OC), lambda bb, rr: (0, 0)),
            pl.BlockSpec((1, OC), lambda bb, rr: (0, 0)),
            pl.BlockSpec((1, OC), lambda bb, rr: (0, 0)),
        ],
        out_specs=pl.BlockSpec((1, OC, RC), lambda bb, rr: (bb, 0, rr)),
        out_shape=jax.ShapeDtypeStruct((B, OC, N), jnp.float32),
    )(h, s1, s2, gamma2, beta2)
    return out


# speedup vs baseline: 13.6950x; 1.1148x over previous
import functools

import jax
import jax.numpy as jnp
from jax.experimental import pallas as pl
from jax.experimental.pallas import tpu as pltpu
from jax.experimental.pallas import tpu_sc as plsc

B, C, N, OC, K = 4, 64, 4096, 128, 16
R = 256
RC = 512

_HIGH = jax.lax.Precision.HIGHEST

NC, NS = 2, 16
NW = NC * NS
PPW = (B * N) // NW
GP = 8
GIDX = GP * K
GPW = PPW // GP


def _knn_topk_kernel(qt_ref, xk_ref, idx_ref):
    qt = qt_ref[0]
    xk = xk_ref[0]

    dot = jax.lax.dot_general(qt, xk, (((1,), (0,)), ((), ())),
                              preferred_element_type=jnp.float32)
    xx = jnp.sum(xk * xk, axis=0, keepdims=True)
    s = 2.0 * dot - xx

    col = jax.lax.broadcasted_iota(jnp.int32, (R, N), 1)
    picks = []
    for _ in range(K):
        m = jnp.max(s, axis=1, keepdims=True)
        idx = jnp.min(jnp.where(s == m, col, N), axis=1,
                      keepdims=True)
        picks.append(idx)
        s = jnp.where(col == idx, -jnp.inf, s)

    idxs = jnp.concatenate(picks, axis=1)
    idx_ref[0] = idxs + pl.program_id(0) * N


def _gather_max_sc(table_hbm, idx_hbm, out_hbm, idx_v, rows_v, out_v,
                   sem0, sem1):
    wid = jax.lax.axis_index("s") * NC + jax.lax.axis_index("c")
    ibase = wid * (PPW * K)
    obase = wid * (PPW // 2)
    sems = (sem0, sem1)

    def issue(c, slot):
        pltpu.sync_copy(idx_hbm.at[pl.ds(ibase + c * GIDX, GIDX)],
                        idx_v.at[slot])
        pltpu.make_async_copy(table_hbm.at[idx_v.at[slot]],
                              rows_v.at[slot], sems[slot]).start()

    def consume(slot):
        pltpu.make_async_copy(table_hbm.at[idx_v.at[slot]],
                              rows_v.at[slot], sems[slot]).wait()
        for p in range(GP):
            for g in range(4):
                r0 = p * K
                acc = rows_v[slot, r0, pl.ds(g * 16, 16)]
                for j in range(1, K):
                    acc = jnp.maximum(
                        acc, rows_v[slot, r0 + j, pl.ds(g * 16, 16)])
                out_v[slot * (GP // 2) + p // 2,
                      pl.ds((p % 2) * C + g * 16, 16)] = acc

    issue(0, 0)

    def body(i, carry):
        c0 = 2 * i
        issue(c0 + 1, 1)
        consume(0)

        @pl.when(i < GPW // 2 - 1)
        def _():
            issue(c0 + 2, 0)

        consume(1)
        pltpu.sync_copy(out_v,
                        out_hbm.at[pl.ds(obase + i * GP, GP)])
        return carry

    jax.lax.fori_loop(0, GPW // 2, body, 0)


def _conv_kernel(xm_ref, w_ref, b_ref, h_ref, sum_ref, sumsq_ref):
    xm = xm_ref[0]
    h = jax.lax.dot_general(xm, w_ref[...], (((1,), (1,)), ((), ())),
                            preferred_element_type=jnp.float32)
    h = h + b_ref[...]
    h_ref[0] = h

    first = jnp.logical_and(pl.program_id(0) == 0, pl.program_id(1) == 0)

    @pl.when(first)
    def _init():
        sum_ref[...] = jnp.zeros((1, OC), jnp.float32)
        sumsq_ref[...] = jnp.zeros((1, OC), jnp.float32)

    sum_ref[...] += jnp.sum(h, axis=0, keepdims=True)
    sumsq_ref[...] += jnp.sum(h * h, axis=0, keepdims=True)


def _bn_relu_kernel(h_ref, sum_ref, sumsq_ref, gamma_ref, beta_ref, out_ref):
    h = h_ref[0]
    cnt = jnp.float32(B * N)
    mean = sum_ref[...] / cnt
    var = sumsq_ref[...] / cnt - mean * mean
    inv = 1.0 / jnp.sqrt(var + 1e-5)
    y = (h - mean) * inv * gamma_ref[...] + beta_ref[...]
    y = jnp.maximum(y, 0.0)
    out_ref[0] = jnp.transpose(y)


def kernel(x, W, b, gamma, beta):
    xt = jnp.transpose(x, (0, 2, 1))
    b2 = b.reshape(1, OC)
    gamma2 = gamma.reshape(1, OC)
    beta2 = beta.reshape(1, OC)

    idx = pl.pallas_call(
        _knn_topk_kernel,
        grid=(B, N // R),
        in_specs=[
            pl.BlockSpec((1, R, C), lambda bb, rr: (bb, rr, 0)),
            pl.BlockSpec((1, C, N), lambda bb, rr: (bb, 0, 0)),
        ],
        out_specs=pl.BlockSpec((1, R, K), lambda bb, rr: (bb, rr, 0)),
        out_shape=jax.ShapeDtypeStruct((B, N, K), jnp.int32),
    )(xt, x)

    table = jnp.concatenate(
        [xt.reshape(B * N, C), jnp.zeros((B * N, C), jnp.float32)], axis=1)
    idx_flat = idx.reshape(B * N * K)

    gather_max = functools.partial(
        pl.kernel,
        mesh=plsc.VectorSubcoreMesh(core_axis_name="c", subcore_axis_name="s"),
        out_type=jax.ShapeDtypeStruct((B * N // 2, 2 * C), jnp.float32),
        scratch_types=[
            pltpu.VMEM((2, GIDX), jnp.int32),
            pltpu.VMEM((2, GIDX, 2 * C), jnp.float32),
            pltpu.VMEM((GP, 2 * C), jnp.float32),
            pltpu.SemaphoreType.DMA,
            pltpu.SemaphoreType.DMA,
        ],
    )(_gather_max_sc)
    xm = gather_max(table, idx_flat).reshape(B, N, C)

    grid = (B, N // RC)
    h, s1, s2 = pl.pallas_call(
        _conv_kernel,
        grid=grid,
        in_specs=[
            pl.BlockSpec((1, RC, C), lambda bb, rr: (bb, rr, 0)),
            pl.BlockSpec((OC, C), lambda bb, rr: (0, 0)),
            pl.BlockSpec((1, OC), lambda bb, rr: (0, 0)),
        ],
        out_specs=[
            pl.BlockSpec((1, RC, OC), lambda bb, rr: (bb, rr, 0)),
            pl.BlockSpec((1, OC), lambda bb, rr: (0, 0)),
            pl.BlockSpec((1, OC), lambda bb, rr: (0, 0)),
        ],
        out_shape=[
            jax.ShapeDtypeStruct((B, N, OC), jnp.float32),
            jax.ShapeDtypeStruct((1, OC), jnp.float32),
            jax.ShapeDtypeStruct((1, OC), jnp.float32),
        ],
    )(xm, W, b2)

    out = pl.pallas_call(
        _bn_relu_kernel,
        grid=grid,
        in_specs=[
            pl.BlockSpec((1, RC, OC), lambda bb, rr: (bb, rr, 0)),
            pl.BlockSpec((1, OC), lambda bb, rr: (0, 0)),
            pl.BlockSpec((1, OC), lambda bb, rr: (0, 0)),
            pl.BlockSpec((1, OC), lambda bb, rr: (0, 0)),
            pl.BlockSpec((1, OC), lambda bb, rr: (0, 0)),
        ],
        out_specs=pl.BlockSpec((1, OC, RC), lambda bb, rr: (bb, 0, rr)),
        out_shape=jax.ShapeDtypeStruct((B, OC, N), jnp.float32),
    )(h, s1, s2, gamma2, beta2)
    return out


# speedup vs baseline: 15.0538x; 1.0992x over previous
import functools

import jax
import jax.numpy as jnp
from jax.experimental import pallas as pl
from jax.experimental.pallas import tpu as pltpu
from jax.experimental.pallas import tpu_sc as plsc

B, C, N, OC, K = 4, 64, 4096, 128, 16
R = 256
RC = 512

_HIGH = jax.lax.Precision.HIGHEST

NC, NS = 2, 16
NW = NC * NS
PPW = (B * N) // NW
GP = 8
GIDX = GP * K
GPW = PPW // GP


def _knn_topk_kernel(qt_ref, xk_ref, idx_ref):
    qt = qt_ref[0]
    xk = xk_ref[0]

    dot = jax.lax.dot_general(qt, xk, (((1,), (0,)), ((), ())),
                              preferred_element_type=jnp.float32)
    xx = jnp.sum(xk * xk, axis=0, keepdims=True)
    s = 2.0 * dot - xx

    col = jax.lax.broadcasted_iota(jnp.int32, (R, N), 1)
    row = jax.lax.broadcasted_iota(jnp.int32, (R, 1), 0) \
        + pl.program_id(1) * R
    s = jnp.where(col == row, -jnp.inf, s)

    H = N // 2
    lo = s[:, :H]
    hi = s[:, H:]
    colh = jax.lax.broadcasted_iota(jnp.int32, (R, H), 1)
    swap = hi > lo
    v = jnp.where(swap, hi, lo)
    vt = jnp.where(swap, lo, hi)
    iv = jnp.where(swap, colh + H, colh)
    ivt = 2 * colh + H - iv
    big = jnp.int32(1 << 30)

    picks = [row]
    for _ in range(K - 1):
        m = jnp.max(v, axis=1, keepdims=True)
        sel = jnp.where(v == m, iv, big)
        idx = jnp.min(sel, axis=1, keepdims=True)
        picks.append(idx)
        onehot = sel == idx
        v = jnp.where(onehot, vt, v)
        iv = jnp.where(onehot, ivt, iv)
        vt = jnp.where(onehot, -jnp.inf, vt)

    idxs = jnp.concatenate(picks, axis=1)
    idx_ref[0] = idxs + pl.program_id(0) * N


def _gather_max_sc(table_hbm, idx_hbm, out_hbm, idx_v, rows_v, out_v,
                   sem0, sem1):
    wid = jax.lax.axis_index("s") * NC + jax.lax.axis_index("c")
    ibase = wid * (PPW * K)
    obase = wid * (PPW // 2)
    sems = (sem0, sem1)

    def issue(c, slot):
        pltpu.sync_copy(idx_hbm.at[pl.ds(ibase + c * GIDX, GIDX)],
                        idx_v.at[slot])
        pltpu.make_async_copy(table_hbm.at[idx_v.at[slot]],
                              rows_v.at[slot], sems[slot]).start()

    def consume(slot):
        pltpu.make_async_copy(table_hbm.at[idx_v.at[slot]],
                              rows_v.at[slot], sems[slot]).wait()
        for p in range(GP):
            for g in range(4):
                r0 = p * K
                acc = rows_v[slot, r0, pl.ds(g * 16, 16)]
                for j in range(1, K):
                    acc = jnp.maximum(
                        acc, rows_v[slot, r0 + j, pl.ds(g * 16, 16)])
                out_v[slot * (GP // 2) + p // 2,
                      pl.ds((p % 2) * C + g * 16, 16)] = acc

    issue(0, 0)

    def body(i, carry):
        c0 = 2 * i
        issue(c0 + 1, 1)
        consume(0)

        @pl.when(i < GPW // 2 - 1)
        def _():
            issue(c0 + 2, 0)

        consume(1)
        pltpu.sync_copy(out_v,
                        out_hbm.at[pl.ds(obase + i * GP, GP)])
        return carry

    jax.lax.fori_loop(0, GPW // 2, body, 0)


def _conv_kernel(xm_ref, w_ref, b_ref, h_ref, sum_ref, sumsq_ref):
    xm = xm_ref[0]
    h = jax.lax.dot_general(xm, w_ref[...], (((1,), (1,)), ((), ())),
                            preferred_element_type=jnp.float32)
    h = h + b_ref[...]
    h_ref[0] = h

    first = jnp.logical_and(pl.program_id(0) == 0, pl.program_id(1) == 0)

    @pl.when(first)
    def _init():
        sum_ref[...] = jnp.zeros((1, OC), jnp.float32)
        sumsq_ref[...] = jnp.zeros((1, OC), jnp.float32)

    sum_ref[...] += jnp.sum(h, axis=0, keepdims=True)
    sumsq_ref[...] += jnp.sum(h * h, axis=0, keepdims=True)


def _bn_relu_kernel(h_ref, sum_ref, sumsq_ref, gamma_ref, beta_ref, out_ref):
    h = h_ref[0]
    cnt = jnp.float32(B * N)
    mean = sum_ref[...] / cnt
    var = sumsq_ref[...] / cnt - mean * mean
    inv = 1.0 / jnp.sqrt(var + 1e-5)
    y = (h - mean) * inv * gamma_ref[...] + beta_ref[...]
    y = jnp.maximum(y, 0.0)
    out_ref[0] = jnp.transpose(y)


def kernel(x, W, b, gamma, beta):
    xt = jnp.transpose(x, (0, 2, 1))
    b2 = b.reshape(1, OC)
    gamma2 = gamma.reshape(1, OC)
    beta2 = beta.reshape(1, OC)

    idx = pl.pallas_call(
        _knn_topk_kernel,
        grid=(B, N // R),
        in_specs=[
            pl.BlockSpec((1, R, C), lambda bb, rr: (bb, rr, 0)),
            pl.BlockSpec((1, C, N), lambda bb, rr: (bb, 0, 0)),
        ],
        out_specs=pl.BlockSpec((1, R, K), lambda bb, rr: (bb, rr, 0)),
        out_shape=jax.ShapeDtypeStruct((B, N, K), jnp.int32),
    )(xt, x)

    table = jnp.concatenate(
        [xt.reshape(B * N, C), jnp.zeros((B * N, C), jnp.float32)], axis=1)
    idx_flat = idx.reshape(B * N * K)

    gather_max = functools.partial(
        pl.kernel,
        mesh=plsc.VectorSubcoreMesh(core_axis_name="c", subcore_axis_name="s"),
        out_type=jax.ShapeDtypeStruct((B * N // 2, 2 * C), jnp.float32),
        scratch_types=[
            pltpu.VMEM((2, GIDX), jnp.int32),
            pltpu.VMEM((2, GIDX, 2 * C), jnp.float32),
            pltpu.VMEM((GP, 2 * C), jnp.float32),
            pltpu.SemaphoreType.DMA,
            pltpu.SemaphoreType.DMA,
        ],
    )(_gather_max_sc)
    xm = gather_max(table, idx_flat).reshape(B, N, C)

    grid = (B, N // RC)
    h, s1, s2 = pl.pallas_call(
        _conv_kernel,
        grid=grid,
        in_specs=[
            pl.BlockSpec((1, RC, C), lambda bb, rr: (bb, rr, 0)),
            pl.BlockSpec((OC, C), lambda bb, rr: (0, 0)),
            pl.BlockSpec((1, OC), lambda bb, rr: (0, 0)),
        ],
        out_specs=[
            pl.BlockSpec((1, RC, OC), lambda bb, rr: (bb, rr, 0)),
            pl.BlockSpec((1, OC), lambda bb, rr: (0, 0)),
            pl.BlockSpec((1, OC), lambda bb, rr: (0, 0)),
        ],
        out_shape=[
            jax.ShapeDtypeStruct((B, N, OC), jnp.float32),
            jax.ShapeDtypeStruct((1, OC), jnp.float32),
            jax.ShapeDtypeStruct((1, OC), jnp.float32),
        ],
    )(xm, W, b2)

    out = pl.pallas_call(
        _bn_relu_kernel,
        grid=grid,
        in_specs=[
            pl.BlockSpec((1, RC, OC), lambda bb, rr: (bb, rr, 0)),
            pl.BlockSpec((1, OC), lambda bb, rr: (0, 0)),
            pl.BlockSpec((1, OC), lambda bb, rr: (0, 0)),
            pl.BlockSpec((1, OC), lambda bb, rr: (0, 0)),
            pl.BlockSpec((1, OC), lambda bb, rr: (0, 0)),
        ],
        out_specs=pl.BlockSpec((1, OC, RC), lambda bb, rr: (bb, 0, rr)),
        out_shape=jax.ShapeDtypeStruct((B, OC, N), jnp.float32),
    )(h, s1, s2, gamma2, beta2)
    return out


# speedup vs baseline: 15.1349x; 1.0054x over previous
import functools

import jax
import jax.numpy as jnp
from jax.experimental import pallas as pl
from jax.experimental.pallas import tpu as pltpu
from jax.experimental.pallas import tpu_sc as plsc

B, C, N, OC, K = 4, 64, 4096, 128, 16
R = 256
RC = 512

_HIGH = jax.lax.Precision.HIGHEST

NC, NS = 2, 16
NW = NC * NS
PPW = (B * N) // NW
GP = 8
GIDX = GP * K
GPW = PPW // GP


def _knn_topk_kernel(qt_ref, xk_ref, idx_ref, tbl_ref):
    qt = qt_ref[0]
    xk = xk_ref[0]
    tbl_ref[0] = jnp.concatenate(
        [qt, jnp.zeros((R, C), jnp.float32)], axis=1)

    dot = jax.lax.dot_general(qt, xk, (((1,), (0,)), ((), ())),
                              preferred_element_type=jnp.float32)
    xx = jnp.sum(xk * xk, axis=0, keepdims=True)
    s = 2.0 * dot - xx

    col = jax.lax.broadcasted_iota(jnp.int32, (R, N), 1)
    row = jax.lax.broadcasted_iota(jnp.int32, (R, 1), 0) \
        + pl.program_id(1) * R
    s = jnp.where(col == row, -jnp.inf, s)

    H = N // 2
    lo = s[:, :H]
    hi = s[:, H:]
    colh = jax.lax.broadcasted_iota(jnp.int32, (R, H), 1)
    swap = hi > lo
    v = jnp.where(swap, hi, lo)
    vt = jnp.where(swap, lo, hi)
    iv = jnp.where(swap, colh + H, colh)
    ivt = 2 * colh + H - iv
    big = jnp.int32(1 << 30)

    picks = [row]
    for _ in range(K - 1):
        m = jnp.max(v, axis=1, keepdims=True)
        sel = jnp.where(v == m, iv, big)
        idx = jnp.min(sel, axis=1, keepdims=True)
        picks.append(idx)
        onehot = sel == idx
        v = jnp.where(onehot, vt, v)
        iv = jnp.where(onehot, ivt, iv)
        vt = jnp.where(onehot, -jnp.inf, vt)

    idxs = jnp.concatenate(picks, axis=1)
    idx_ref[0] = idxs + pl.program_id(0) * N


def _gather_max_sc(table_hbm, idx_hbm, out_hbm, idx_v, rows_v, out_v,
                   sem0, sem1):
    wid = jax.lax.axis_index("s") * NC + jax.lax.axis_index("c")
    ibase = wid * (PPW * K)
    obase = wid * (PPW // 2)
    sems = (sem0, sem1)

    def issue(c, slot):
        pltpu.sync_copy(idx_hbm.at[pl.ds(ibase + c * GIDX, GIDX)],
                        idx_v.at[slot])
        pltpu.make_async_copy(table_hbm.at[idx_v.at[slot]],
                              rows_v.at[slot], sems[slot]).start()

    def consume(slot):
        pltpu.make_async_copy(table_hbm.at[idx_v.at[slot]],
                              rows_v.at[slot], sems[slot]).wait()
        for p in range(GP):
            for g in range(4):
                r0 = p * K
                acc = rows_v[slot, r0, pl.ds(g * 16, 16)]
                for j in range(1, K):
                    acc = jnp.maximum(
                        acc, rows_v[slot, r0 + j, pl.ds(g * 16, 16)])
                out_v[slot * (GP // 2) + p // 2,
                      pl.ds((p % 2) * C + g * 16, 16)] = acc

    issue(0, 0)

    def body(i, carry):
        c0 = 2 * i
        issue(c0 + 1, 1)
        consume(0)

        @pl.when(i < GPW // 2 - 1)
        def _():
            issue(c0 + 2, 0)

        consume(1)
        pltpu.sync_copy(out_v,
                        out_hbm.at[pl.ds(obase + i * GP, GP)])
        return carry

    jax.lax.fori_loop(0, GPW // 2, body, 0)


def _conv_kernel(xm_ref, w_ref, b_ref, h_ref, sum_ref, sumsq_ref):
    xm = xm_ref[0]
    h = jax.lax.dot_general(xm, w_ref[...], (((1,), (1,)), ((), ())),
                            preferred_element_type=jnp.float32)
    h = h + b_ref[...]
    h_ref[0] = h

    first = jnp.logical_and(pl.program_id(0) == 0, pl.program_id(1) == 0)

    @pl.when(first)
    def _init():
        sum_ref[...] = jnp.zeros((1, OC), jnp.float32)
        sumsq_ref[...] = jnp.zeros((1, OC), jnp.float32)

    sum_ref[...] += jnp.sum(h, axis=0, keepdims=True)
    sumsq_ref[...] += jnp.sum(h * h, axis=0, keepdims=True)


def _bn_relu_kernel(h_ref, sum_ref, sumsq_ref, gamma_ref, beta_ref, out_ref):
    h = h_ref[0]
    cnt = jnp.float32(B * N)
    mean = sum_ref[...] / cnt
    var = sumsq_ref[...] / cnt - mean * mean
    inv = 1.0 / jnp.sqrt(var + 1e-5)
    y = (h - mean) * inv * gamma_ref[...] + beta_ref[...]
    y = jnp.maximum(y, 0.0)
    out_ref[0] = jnp.transpose(y)


def kernel(x, W, b, gamma, beta):
    xt = jnp.transpose(x, (0, 2, 1))
    b2 = b.reshape(1, OC)
    gamma2 = gamma.reshape(1, OC)
    beta2 = beta.reshape(1, OC)

    idx, tbl = pl.pallas_call(
        _knn_topk_kernel,
        grid=(B, N // R),
        in_specs=[
            pl.BlockSpec((1, R, C), lambda bb, rr: (bb, rr, 0)),
            pl.BlockSpec((1, C, N), lambda bb, rr: (bb, 0, 0)),
        ],
        out_specs=[
            pl.BlockSpec((1, R, K), lambda bb, rr: (bb, rr, 0)),
            pl.BlockSpec((1, R, 2 * C), lambda bb, rr: (bb, rr, 0)),
        ],
        out_shape=[
            jax.ShapeDtypeStruct((B, N, K), jnp.int32),
            jax.ShapeDtypeStruct((B, N, 2 * C), jnp.float32),
        ],
    )(xt, x)

    table = tbl.reshape(B * N, 2 * C)
    idx_flat = idx.reshape(B * N * K)

    gather_max = functools.partial(
        pl.kernel,
        mesh=plsc.VectorSubcoreMesh(core_axis_name="c", subcore_axis_name="s"),
        out_type=jax.ShapeDtypeStruct((B * N // 2, 2 * C), jnp.float32),
        scratch_types=[
            pltpu.VMEM((2, GIDX), jnp.int32),
            pltpu.VMEM((2, GIDX, 2 * C), jnp.float32),
            pltpu.VMEM((GP, 2 * C), jnp.float32),
            pltpu.SemaphoreType.DMA,
            pltpu.SemaphoreType.DMA,
        ],
    )(_gather_max_sc)
    xm = gather_max(table, idx_flat).reshape(B, N, C)

    grid = (B, N // RC)
    h, s1, s2 = pl.pallas_call(
        _conv_kernel,
        grid=grid,
        in_specs=[
            pl.BlockSpec((1, RC, C), lambda bb, rr: (bb, rr, 0)),
            pl.BlockSpec((OC, C), lambda bb, rr: (0, 0)),
            pl.BlockSpec((1, OC), lambda bb, rr: (0, 0)),
        ],
        out_specs=[
            pl.BlockSpec((1, RC, OC), lambda bb, rr: (bb, rr, 0)),
            pl.BlockSpec((1, OC), lambda bb, rr: (0, 0)),
            pl.BlockSpec((1, OC), lambda bb, rr: (0, 0)),
        ],
        out_shape=[
            jax.ShapeDtypeStruct((B, N, OC), jnp.float32),
            jax.ShapeDtypeStruct((1, OC), jnp.float32),
            jax.ShapeDtypeStruct((1, OC), jnp.float32),
        ],
    )(xm, W, b2)

    out = pl.pallas_call(
        _bn_relu_kernel,
        grid=grid,
        in_specs=[
            pl.BlockSpec((1, RC, OC), lambda bb, rr: (bb, rr, 0)),
            pl.BlockSpec((1, OC), lambda bb, rr: (0, 0)),
            pl.BlockSpec((1, OC), lambda bb, rr: (0, 0)),
            pl.BlockSpec((1, OC), lambda bb, rr: (0, 0)),
            pl.BlockSpec((1, OC), lambda bb, rr: (0, 0)),
        ],
        out_specs=pl.BlockSpec((1, OC, RC), lambda bb, rr: (bb, 0, rr)),
        out_shape=jax.ShapeDtypeStruct((B, OC, N), jnp.float32),
    )(h, s1, s2, gamma2, beta2)
    return out


# speedup vs baseline: 16.4585x; 1.0875x over previous
import functools

import jax
import jax.numpy as jnp
from jax.experimental import pallas as pl
from jax.experimental.pallas import tpu as pltpu
from jax.experimental.pallas import tpu_sc as plsc

B, C, N, OC, K = 4, 64, 4096, 128, 16
R = 256
RC = 512

_HIGH = jax.lax.Precision.HIGHEST

NC, NS = 2, 16
NW = NC * NS
PPW = (B * N) // NW
GP = 8
GIDX = GP * K
GPW = PPW // GP


def _knn_topk_kernel(qt_ref, xk_ref, idx_ref, tbl_ref):
    qt = qt_ref[0]
    xk = xk_ref[0]
    tbl_ref[0] = jnp.concatenate(
        [qt, jnp.zeros((R, C), jnp.float32)], axis=1)

    dot = jax.lax.dot_general(qt, xk, (((1,), (0,)), ((), ())),
                              preferred_element_type=jnp.float32)
    xx = jnp.sum(xk * xk, axis=0, keepdims=True)
    s = 2.0 * dot - xx

    col = jax.lax.broadcasted_iota(jnp.int32, (R, N), 1)
    row = jax.lax.broadcasted_iota(jnp.int32, (R, 1), 0) \
        + pl.program_id(1) * R
    s = jnp.where(col == row, -jnp.inf, s)

    H = N // 2
    lo = s[:, :H]
    hi = s[:, H:]
    colh = jax.lax.broadcasted_iota(jnp.int32, (R, H), 1).astype(jnp.float32)
    swap = hi > lo
    v = jnp.where(swap, hi, lo)
    vt = jnp.where(swap, lo, hi)
    iv = jnp.where(swap, colh + H, colh)
    ivt = 2.0 * colh + H - iv
    big = jnp.float32(1 << 24)

    picks = [row.astype(jnp.float32)]
    for _ in range(K - 1):
        m = jnp.max(v, axis=1, keepdims=True)
        sel = jnp.where(v == m, iv, big)
        idx = jnp.min(sel, axis=1, keepdims=True)
        picks.append(idx)
        onehot = sel == idx
        v = jnp.where(onehot, vt, v)
        iv = jnp.where(onehot, ivt, iv)
        vt = jnp.where(onehot, -jnp.inf, vt)

    idxs = jnp.concatenate(picks, axis=1).astype(jnp.int32)
    idx_ref[0] = idxs + pl.program_id(0) * N


def _gather_max_sc(table_hbm, idx_hbm, out_hbm, idx_v, rows_v, out_v,
                   sem0, sem1):
    wid = jax.lax.axis_index("s") * NC + jax.lax.axis_index("c")
    ibase = wid * (PPW * K)
    obase = wid * (PPW // 2)
    sems = (sem0, sem1)

    def issue(c, slot):
        pltpu.sync_copy(idx_hbm.at[pl.ds(ibase + c * GIDX, GIDX)],
                        idx_v.at[slot])
        pltpu.make_async_copy(table_hbm.at[idx_v.at[slot]],
                              rows_v.at[slot], sems[slot]).start()

    def consume(slot):
        pltpu.make_async_copy(table_hbm.at[idx_v.at[slot]],
                              rows_v.at[slot], sems[slot]).wait()
        for p in range(GP):
            for g in range(4):
                r0 = p * K
                acc = rows_v[slot, r0, pl.ds(g * 16, 16)]
                for j in range(1, K):
                    acc = jnp.maximum(
                        acc, rows_v[slot, r0 + j, pl.ds(g * 16, 16)])
                out_v[slot * (GP // 2) + p // 2,
                      pl.ds((p % 2) * C + g * 16, 16)] = acc

    issue(0, 0)

    def body(i, carry):
        c0 = 2 * i
        issue(c0 + 1, 1)
        consume(0)

        @pl.when(i < GPW // 2 - 1)
        def _():
            issue(c0 + 2, 0)

        consume(1)
        pltpu.sync_copy(out_v,
                        out_hbm.at[pl.ds(obase + i * GP, GP)])
        return carry

    jax.lax.fori_loop(0, GPW // 2, body, 0)


def _conv_kernel(xm_ref, w_ref, b_ref, h_ref, sum_ref, sumsq_ref):
    xm = xm_ref[0]
    h = jax.lax.dot_general(xm, w_ref[...], (((1,), (1,)), ((), ())),
                            preferred_element_type=jnp.float32)
    h = h + b_ref[...]
    h_ref[0] = h

    first = jnp.logical_and(pl.program_id(0) == 0, pl.program_id(1) == 0)

    @pl.when(first)
    def _init():
        sum_ref[...] = jnp.zeros((1, OC), jnp.float32)
        sumsq_ref[...] = jnp.zeros((1, OC), jnp.float32)

    sum_ref[...] += jnp.sum(h, axis=0, keepdims=True)
    sumsq_ref[...] += jnp.sum(h * h, axis=0, keepdims=True)


def _bn_relu_kernel(h_ref, sum_ref, sumsq_ref, gamma_ref, beta_ref, out_ref):
    h = h_ref[0]
    cnt = jnp.float32(B * N)
    mean = sum_ref[...] / cnt
    var = sumsq_ref[...] / cnt - mean * mean
    inv = 1.0 / jnp.sqrt(var + 1e-5)
    y = (h - mean) * inv * gamma_ref[...] + beta_ref[...]
    y = jnp.maximum(y, 0.0)
    out_ref[0] = jnp.transpose(y)


def kernel(x, W, b, gamma, beta):
    xt = jnp.transpose(x, (0, 2, 1))
    b2 = b.reshape(1, OC)
    gamma2 = gamma.reshape(1, OC)
    beta2 = beta.reshape(1, OC)

    idx, tbl = pl.pallas_call(
        _knn_topk_kernel,
        grid=(B, N // R),
        in_specs=[
            pl.BlockSpec((1, R, C), lambda bb, rr: (bb, rr, 0)),
            pl.BlockSpec((1, C, N), lambda bb, rr: (bb, 0, 0)),
        ],
        out_specs=[
            pl.BlockSpec((1, R, K), lambda bb, rr: (bb, rr, 0)),
            pl.BlockSpec((1, R, 2 * C), lambda bb, rr: (bb, rr, 0)),
        ],
        out_shape=[
            jax.ShapeDtypeStruct((B, N, K), jnp.int32),
            jax.ShapeDtypeStruct((B, N, 2 * C), jnp.float32),
        ],
    )(xt, x)

    table = tbl.reshape(B * N, 2 * C)
    idx_flat = idx.reshape(B * N * K)

    gather_max = functools.partial(
        pl.kernel,
        mesh=plsc.VectorSubcoreMesh(core_axis_name="c", subcore_axis_name="s"),
        out_type=jax.ShapeDtypeStruct((B * N // 2, 2 * C), jnp.float32),
        scratch_types=[
            pltpu.VMEM((2, GIDX), jnp.int32),
            pltpu.VMEM((2, GIDX, 2 * C), jnp.float32),
            pltpu.VMEM((GP, 2 * C), jnp.float32),
            pltpu.SemaphoreType.DMA,
            pltpu.SemaphoreType.DMA,
        ],
    )(_gather_max_sc)
    xm = gather_max(table, idx_flat).reshape(B, N, C)

    grid = (B, N // RC)
    h, s1, s2 = pl.pallas_call(
        _conv_kernel,
        grid=grid,
        in_specs=[
            pl.BlockSpec((1, RC, C), lambda bb, rr: (bb, rr, 0)),
            pl.BlockSpec((OC, C), lambda bb, rr: (0, 0)),
            pl.BlockSpec((1, OC), lambda bb, rr: (0, 0)),
        ],
        out_specs=[
            pl.BlockSpec((1, RC, OC), lambda bb, rr: (bb, rr, 0)),
            pl.BlockSpec((1, OC), lambda bb, rr: (0, 0)),
            pl.BlockSpec((1, OC), lambda bb, rr: (0, 0)),
        ],
        out_shape=[
            jax.ShapeDtypeStruct((B, N, OC), jnp.float32),
            jax.ShapeDtypeStruct((1, OC), jnp.float32),
            jax.ShapeDtypeStruct((1, OC), jnp.float32),
        ],
    )(xm, W, b2)

    out = pl.pallas_call(
        _bn_relu_kernel,
        grid=grid,
        in_specs=[
            pl.BlockSpec((1, RC, OC), lambda bb, rr: (bb, rr, 0)),
            pl.BlockSpec((1, OC), lambda bb, rr: (0, 0)),
            pl.BlockSpec((1, OC), lambda bb, rr: (0, 0)),
            pl.BlockSpec((1, OC), lambda bb, rr: (0, 0)),
            pl.BlockSpec((1, OC), lambda bb, rr: (0, 0)),
        ],
        out_specs=pl.BlockSpec((1, OC, RC), lambda bb, rr: (bb, 0, rr)),
        out_shape=jax.ShapeDtypeStruct((B, OC, N), jnp.float32),
    )(h, s1, s2, gamma2, beta2)
    return out


# speedup vs baseline: 17.5093x; 1.0638x over previous
import functools

import jax
import jax.numpy as jnp
from jax.experimental import pallas as pl
from jax.experimental.pallas import tpu as pltpu
from jax.experimental.pallas import tpu_sc as plsc

B, C, N, OC, K = 4, 64, 4096, 128, 16
R = 256
RC = 512

_HIGH = jax.lax.Precision.HIGHEST

NC, NS = 2, 16
NW = NC * NS
PPW = N // NW
GP = 8
GIDX = GP * K
GPW = PPW // GP


def _knn_topk_kernel(qt_ref, xk_ref, idx_ref, tbl_ref):
    qt = qt_ref[...]
    xk = xk_ref[...]
    tbl_ref[...] = jnp.concatenate(
        [qt, jnp.zeros((R, C), jnp.float32)], axis=1)

    dot = jax.lax.dot_general(qt, xk, (((1,), (0,)), ((), ())),
                              preferred_element_type=jnp.float32)
    xx = jnp.sum(xk * xk, axis=0, keepdims=True)
    s = 2.0 * dot - xx

    col = jax.lax.broadcasted_iota(jnp.int32, (R, N), 1)
    row = jax.lax.broadcasted_iota(jnp.int32, (R, 1), 0) \
        + pl.program_id(0) * R
    s = jnp.where(col == row, -jnp.inf, s)

    H = N // 2
    lo = s[:, :H]
    hi = s[:, H:]
    colh = jax.lax.broadcasted_iota(jnp.int32, (R, H), 1).astype(jnp.float32)
    swap = hi > lo
    v = jnp.where(swap, hi, lo)
    vt = jnp.where(swap, lo, hi)
    iv = jnp.where(swap, colh + H, colh)
    ivt = 2.0 * colh + H - iv
    big = jnp.float32(1 << 24)

    picks = [row.astype(jnp.float32)]
    for _ in range(K - 1):
        m = jnp.max(v, axis=1, keepdims=True)
        sel = jnp.where(v == m, iv, big)
        idx = jnp.min(sel, axis=1, keepdims=True)
        picks.append(idx)
        onehot = sel == idx
        v = jnp.where(onehot, vt, v)
        iv = jnp.where(onehot, ivt, iv)
        vt = jnp.where(onehot, -jnp.inf, vt)

    idxs = jnp.concatenate(picks, axis=1).astype(jnp.int32)
    idx_ref[...] = idxs


def _gather_max_sc(table_hbm, idx_hbm, out_hbm, idx_v, rows_v, out_v,
                   sem0, sem1):
    wid = jax.lax.axis_index("s") * NC + jax.lax.axis_index("c")
    ibase = wid * (PPW * K)
    obase = wid * (PPW // 2)
    sems = (sem0, sem1)

    def issue(c, slot):
        pltpu.sync_copy(idx_hbm.at[pl.ds(ibase + c * GIDX, GIDX)],
                        idx_v.at[slot])
        pltpu.make_async_copy(table_hbm.at[idx_v.at[slot]],
                              rows_v.at[slot], sems[slot]).start()

    def consume(slot):
        pltpu.make_async_copy(table_hbm.at[idx_v.at[slot]],
                              rows_v.at[slot], sems[slot]).wait()
        for p in range(GP):
            for g in range(4):
                r0 = p * K
                acc = rows_v[slot, r0, pl.ds(g * 16, 16)]
                for j in range(1, K):
                    acc = jnp.maximum(
                        acc, rows_v[slot, r0 + j, pl.ds(g * 16, 16)])
                out_v[slot * (GP // 2) + p // 2,
                      pl.ds((p % 2) * C + g * 16, 16)] = acc

    issue(0, 0)

    def body(i, carry):
        c0 = 2 * i
        issue(c0 + 1, 1)
        consume(0)

        @pl.when(i < GPW // 2 - 1)
        def _():
            issue(c0 + 2, 0)

        consume(1)
        pltpu.sync_copy(out_v,
                        out_hbm.at[pl.ds(obase + i * GP, GP)])
        return carry

    jax.lax.fori_loop(0, GPW // 2, body, 0)


def _conv_kernel(xm_ref, w_ref, b_ref, h_ref, sum_ref, sumsq_ref):
    xm = xm_ref[0]
    h = jax.lax.dot_general(xm, w_ref[...], (((1,), (1,)), ((), ())),
                            preferred_element_type=jnp.float32)
    h = h + b_ref[...]
    h_ref[0] = h

    first = jnp.logical_and(pl.program_id(0) == 0, pl.program_id(1) == 0)

    @pl.when(first)
    def _init():
        sum_ref[...] = jnp.zeros((1, OC), jnp.float32)
        sumsq_ref[...] = jnp.zeros((1, OC), jnp.float32)

    sum_ref[...] += jnp.sum(h, axis=0, keepdims=True)
    sumsq_ref[...] += jnp.sum(h * h, axis=0, keepdims=True)


def _bn_relu_kernel(h_ref, sum_ref, sumsq_ref, gamma_ref, beta_ref, out_ref):
    h = h_ref[0]
    cnt = jnp.float32(B * N)
    mean = sum_ref[...] / cnt
    var = sumsq_ref[...] / cnt - mean * mean
    inv = 1.0 / jnp.sqrt(var + 1e-5)
    y = (h - mean) * inv * gamma_ref[...] + beta_ref[...]
    y = jnp.maximum(y, 0.0)
    out_ref[0] = jnp.transpose(y)


def kernel(x, W, b, gamma, beta):
    xt = jnp.transpose(x, (0, 2, 1))
    b2 = b.reshape(1, OC)
    gamma2 = gamma.reshape(1, OC)
    beta2 = beta.reshape(1, OC)

    topk = pl.pallas_call(
        _knn_topk_kernel,
        grid=(N // R,),
        in_specs=[
            pl.BlockSpec((R, C), lambda rr: (rr, 0)),
            pl.BlockSpec((C, N), lambda rr: (0, 0)),
        ],
        out_specs=[
            pl.BlockSpec((R, K), lambda rr: (rr, 0)),
            pl.BlockSpec((R, 2 * C), lambda rr: (rr, 0)),
        ],
        out_shape=[
            jax.ShapeDtypeStruct((N, K), jnp.int32),
            jax.ShapeDtypeStruct((N, 2 * C), jnp.float32),
        ],
    )

    gather_max = functools.partial(
        pl.kernel,
        mesh=plsc.VectorSubcoreMesh(core_axis_name="c", subcore_axis_name="s"),
        out_type=jax.ShapeDtypeStruct((N // 2, 2 * C), jnp.float32),
        scratch_types=[
            pltpu.VMEM((2, GIDX), jnp.int32),
            pltpu.VMEM((2, GIDX, 2 * C), jnp.float32),
            pltpu.VMEM((GP, 2 * C), jnp.float32),
            pltpu.SemaphoreType.DMA,
            pltpu.SemaphoreType.DMA,
        ],
    )(_gather_max_sc)

    xms = []
    for bb in range(B):
        idx_b, tbl_b = topk(xt[bb], x[bb])
        xms.append(gather_max(tbl_b, idx_b.reshape(N * K)))
    xm = jnp.stack(xms).reshape(B, N, C)

    grid = (B, N // RC)
    h, s1, s2 = pl.pallas_call(
        _conv_kernel,
        grid=grid,
        in_specs=[
            pl.BlockSpec((1, RC, C), lambda bb, rr: (bb, rr, 0)),
            pl.BlockSpec((OC, C), lambda bb, rr: (0, 0)),
            pl.BlockSpec((1, OC), lambda bb, rr: (0, 0)),
        ],
        out_specs=[
            pl.BlockSpec((1, RC, OC), lambda bb, rr: (bb, rr, 0)),
            pl.BlockSpec((1, OC), lambda bb, rr: (0, 0)),
            pl.BlockSpec((1, OC), lambda bb, rr: (0, 0)),
        ],
        out_shape=[
            jax.ShapeDtypeStruct((B, N, OC), jnp.float32),
            jax.ShapeDtypeStruct((1, OC), jnp.float32),
            jax.ShapeDtypeStruct((1, OC), jnp.float32),
        ],
    )(xm, W, b2)

    out = pl.pallas_call(
        _bn_relu_kernel,
        grid=grid,
        in_specs=[
            pl.BlockSpec((1, RC, OC), lambda bb, rr: (bb, rr, 0)),
            pl.BlockSpec((1, OC), lambda bb, rr: (0, 0)),
            pl.BlockSpec((1, OC), lambda bb, rr: (0, 0)),
            pl.BlockSpec((1, OC), lambda bb, rr: (0, 0)),
            pl.BlockSpec((1, OC), lambda bb, rr: (0, 0)),
        ],
        out_specs=pl.BlockSpec((1, OC, RC), lambda bb, rr: (bb, 0, rr)),
        out_shape=jax.ShapeDtypeStruct((B, OC, N), jnp.float32),
    )(h, s1, s2, gamma2, beta2)
    return out
